# baseline (speedup 1.0000x reference)
import functools
import math

import jax
import jax.numpy as jnp
from jax import lax
from jax.experimental import pallas as pl
from jax.experimental.pallas import tpu as pltpu
from jax.experimental.pallas import tpu_sc as plsc

EMB = 64
SCALE = math.sqrt(float(EMB))
NC = 2
NS = 16
NW = NC * NS
LANES = 16


def _xlane_sum(x):
    for d in (1, 2, 4, 8):
        perm = jnp.arange(LANES, dtype=jnp.int32) ^ d
        x = x + x.at[perm].get(mode="promise_in_bounds")
    return x


def _norm_scale(vs):
    v0, v1, v2, v3 = vs
    ss = v0 * v0 + v1 * v1 + v2 * v2 + v3 * v3
    tv = _xlane_sum(ss)
    m = jnp.maximum(tv, 1.0)
    i = lax.bitcast_convert_type(m, jnp.int32)
    i = jnp.int32(0x5F3759DF) - lax.shift_right_arithmetic(i, 1)
    y = lax.bitcast_convert_type(i, jnp.float32)
    h = 0.5 * m
    y = y * (1.5 - h * y * y)
    y = y * (1.5 - h * y * y)
    return y * SCALE


def _row_update(rows_v, r, col):
    vs = [rows_v[r, pl.ds(col + 16 * k, 16)] for k in range(4)]
    f = _norm_scale(vs)
    for k in range(4):
        rows_v[r, pl.ds(col + 16 * k, 16)] = vs[k] * f


@functools.partial(jax.jit, static_argnames=("n", "chunk"))
def _emb_lookup(tokens_perm, table, *, n, chunk):
    per_w = n // NW
    nchunk = per_w // chunk
    c2 = chunk // 2

    mesh = plsc.VectorSubcoreMesh(core_axis_name="c", subcore_axis_name="s")

    @functools.partial(
        pl.kernel,
        mesh=mesh,
        compiler_params=pltpu.CompilerParams(use_tc_tiling_on_sc=False),
        out_type=jax.ShapeDtypeStruct((n // 2, 2 * EMB), jnp.float32),
        scratch_types=[
            pltpu.VMEM((chunk,), jnp.int32),
            pltpu.VMEM((c2, EMB), jnp.float32),
            pltpu.VMEM((c2, EMB), jnp.float32),
            pltpu.SemaphoreType.DMA,
        ],
    )
    def body(tok_hbm, table_hbm, out_hbm, idx_v, rows_a, rows_b, sem):
        wid = lax.axis_index("s") * NC + lax.axis_index("c")
        base = wid * per_w

        def do_chunk(g, carry):
            off = base + g * chunk
            pltpu.sync_copy(tok_hbm.at[pl.ds(off, chunk)], idx_v)
            cpa = pltpu.async_copy(
                table_hbm.at[idx_v.at[pl.ds(0, c2)]], rows_a, sem
            )
            cpb = pltpu.async_copy(
                table_hbm.at[idx_v.at[pl.ds(c2, c2)]], rows_b, sem
            )
            cpa.wait()
            cpb.wait()

            @plsc.parallel_loop(0, c2, unroll=8)
            def do_row(r):
                _row_update(rows_a, r, 0)
                _row_update(rows_b, r, 0)

            out_rows = out_hbm.at[pl.ds(off // 2, c2)]
            pltpu.sync_copy(rows_a, out_rows.at[:, pl.ds(0, EMB)])
            pltpu.sync_copy(rows_b, out_rows.at[:, pl.ds(EMB, EMB)])
            return carry

        lax.fori_loop(0, nchunk, do_chunk, 0)

    return body(tokens_perm, table)


def kernel(tokens, table):
    b, l = tokens.shape
    n = b * l
    chunk = 512
    flat = tokens.reshape(-1).astype(jnp.int32)
    perm = flat.reshape(n // chunk, chunk // 2, 2).transpose(0, 2, 1).reshape(-1)
    out = _emb_lookup(perm, table, n=n, chunk=chunk)
    return out.reshape(b, l, EMB)


# speedup vs baseline: 1.0950x; 1.0950x over previous
import functools
import math

import jax
import jax.numpy as jnp
from jax import lax
from jax.experimental import pallas as pl
from jax.experimental.pallas import tpu as pltpu
from jax.experimental.pallas import tpu_sc as plsc

EMB = 64
SCALE = math.sqrt(float(EMB))
NC = 2
NS = 16
NW = NC * NS
LANES = 16
BCHUNK = 16


def _xlane_sum(x):
    for d in (1, 2, 4, 8):
        perm = jnp.arange(LANES, dtype=jnp.int32) ^ d
        x = x + x.at[perm].get(mode="promise_in_bounds")
    return x


def _row_update(rows_v, r):
    vs = [rows_v[r, pl.ds(16 * k, 16)] for k in range(4)]
    v0, v1, v2, v3 = vs
    ss = v0 * v0 + v1 * v1 + v2 * v2 + v3 * v3
    tv = _xlane_sum(ss)
    m = jnp.maximum(tv, 1.0)
    i = lax.bitcast_convert_type(m, jnp.int32)
    i = jnp.int32(0x5F3759DF) - lax.shift_right_arithmetic(i, 1)
    y = lax.bitcast_convert_type(i, jnp.float32)
    h = 0.5 * m
    y = y * (1.5 - h * y * y)
    y = y * (1.5 - h * y * y)
    f = y * SCALE
    for k in range(4):
        rows_v[r, pl.ds(16 * k, 16)] = vs[k] * f


@functools.partial(jax.jit, static_argnames=("b", "l"))
def _emb_lookup(tokens_flat, table, *, b, l):
    n = b * l
    per_w = n // NW
    b_per_w = b // NW
    nchunk = b_per_w // BCHUNK
    chunk = BCHUNK * l

    mesh = plsc.VectorSubcoreMesh(core_axis_name="c", subcore_axis_name="s")

    @functools.partial(
        pl.kernel,
        mesh=mesh,
        compiler_params=pltpu.CompilerParams(use_tc_tiling_on_sc=False),
        out_type=jax.ShapeDtypeStruct((b, l, EMB), jnp.float32),
        scratch_types=[
            pltpu.VMEM((chunk,), jnp.int32),
            pltpu.VMEM((chunk, EMB), jnp.float32),
            pltpu.SemaphoreType.DMA,
        ],
    )
    def body(tok_hbm, table_hbm, out_hbm, idx_v, rows_v, sem):
        wid = lax.axis_index("s") * NC + lax.axis_index("c")
        base = wid * per_w
        bbase = wid * b_per_w

        def do_chunk(g, carry):
            off = base + g * chunk
            b0 = bbase + g * BCHUNK
            pltpu.sync_copy(tok_hbm.at[pl.ds(off, chunk)], idx_v)
            pltpu.async_copy(table_hbm.at[idx_v], rows_v, sem).wait()

            @plsc.parallel_loop(0, chunk, unroll=8)
            def do_row(r):
                _row_update(rows_v, r)

            for bb in range(BCHUNK):
                pltpu.sync_copy(
                    rows_v.at[pl.ds(bb * l, l)], out_hbm.at[b0 + bb]
                )
            return carry

        lax.fori_loop(0, nchunk, do_chunk, 0)

    return body(tokens_flat, table)


def kernel(tokens, table):
    b, l = tokens.shape
    flat = tokens.reshape(-1).astype(jnp.int32)
    return _emb_lookup(flat, table, b=b, l=l)


# speedup vs baseline: 1.1061x; 1.0101x over previous
import functools
import math

import jax
import jax.numpy as jnp
from jax import lax
from jax.experimental import pallas as pl
from jax.experimental.pallas import tpu as pltpu
from jax.experimental.pallas import tpu_sc as plsc

EMB = 64
SCALE = math.sqrt(float(EMB))
NC = 2
NS = 16
NW = NC * NS
LANES = 16
BCHUNK = 8


def _xlane_sum(x):
    for d in (1, 2, 4, 8):
        perm = jnp.arange(LANES, dtype=jnp.int32) ^ d
        x = x + x.at[perm].get(mode="promise_in_bounds")
    return x


def _row_update(rows_v, out_v, tok_v, r):
    col = (tok_v[pl.ds(r, LANES)][0] & 1) * EMB
    vs = [rows_v[r, pl.ds(col + 16 * k, 16)] for k in range(4)]
    v0, v1, v2, v3 = vs
    ss = v0 * v0 + v1 * v1 + v2 * v2 + v3 * v3
    tv = _xlane_sum(ss)
    m = jnp.maximum(tv, 1.0)
    i = lax.bitcast_convert_type(m, jnp.int32)
    i = jnp.int32(0x5F3759DF) - lax.shift_right_arithmetic(i, 1)
    y = lax.bitcast_convert_type(i, jnp.float32)
    h = 0.5 * m
    y = y * (1.5 - h * y * y)
    y = y * (1.5 - h * y * y)
    f = y * SCALE
    for k in range(4):
        out_v[r, pl.ds(16 * k, 16)] = vs[k] * f


@functools.partial(jax.jit, static_argnames=("b", "l"))
def _emb_lookup(tokens_flat, table2, *, b, l):
    n = b * l
    per_w = n // NW
    b_per_w = b // NW
    nchunk = b_per_w // BCHUNK
    chunk = BCHUNK * l

    mesh = plsc.VectorSubcoreMesh(core_axis_name="c", subcore_axis_name="s")

    @functools.partial(
        pl.kernel,
        mesh=mesh,
        out_type=jax.ShapeDtypeStruct((b, l, EMB), jnp.float32),
        scratch_types=[
            pltpu.VMEM((chunk + LANES,), jnp.int32),
            pltpu.VMEM((chunk,), jnp.int32),
            pltpu.VMEM((chunk, 2 * EMB), jnp.float32),
            pltpu.VMEM((chunk, EMB), jnp.float32),
            pltpu.SemaphoreType.DMA,
        ],
    )
    def body(tok_hbm, table_hbm, out_hbm, tok_v, idx_v, rows_v, out_v, sem):
        wid = lax.axis_index("s") * NC + lax.axis_index("c")
        base = wid * per_w
        bbase = wid * b_per_w

        def do_chunk(g, carry):
            off = base + g * chunk
            b0 = bbase + g * BCHUNK
            pltpu.sync_copy(
                tok_hbm.at[pl.ds(off, chunk)], tok_v.at[pl.ds(0, chunk)]
            )

            @plsc.parallel_loop(0, chunk // LANES, unroll=4)
            def make_idx(i):
                t = tok_v[pl.ds(i * LANES, LANES)]
                idx_v[pl.ds(i * LANES, LANES)] = lax.shift_right_logical(t, 1)

            pltpu.async_copy(table_hbm.at[idx_v], rows_v, sem).wait()

            @plsc.parallel_loop(0, chunk, unroll=8)
            def do_row(r):
                _row_update(rows_v, out_v, tok_v, r)

            for bb in range(BCHUNK):
                pltpu.sync_copy(
                    out_v.at[pl.ds(bb * l, l)], out_hbm.at[b0 + bb]
                )
            return carry

        lax.fori_loop(0, nchunk, do_chunk, 0)

    return body(tokens_flat, table2)


def kernel(tokens, table):
    b, l = tokens.shape
    flat = tokens.reshape(-1).astype(jnp.int32)
    table2 = table.reshape(table.shape[0] // 2, 2 * EMB)
    return _emb_lookup(flat, table2, b=b, l=l)


# speedup vs baseline: 1.2365x; 1.1179x over previous
import functools
import math

import jax
import jax.numpy as jnp
from jax import lax
from jax.experimental import pallas as pl
from jax.experimental.pallas import tpu as pltpu
from jax.experimental.pallas import tpu_sc as plsc

EMB = 64
SCALE = math.sqrt(float(EMB))
NC = 2
NS = 16
NW = NC * NS
LANES = 16
BCHUNK = 4


def _xlane_sum(x):
    for d in (1, 2, 4, 8):
        perm = jnp.arange(LANES, dtype=jnp.int32) ^ d
        x = x + x.at[perm].get(mode="promise_in_bounds")
    return x


def _row_update(rows_v, out_v, tok_v, r):
    col = (tok_v[pl.ds(r, LANES)][0] & 1) * EMB
    vs = [rows_v[r, pl.ds(col + 16 * k, 16)] for k in range(4)]
    v0, v1, v2, v3 = vs
    ss = v0 * v0 + v1 * v1 + v2 * v2 + v3 * v3
    tv = _xlane_sum(ss)
    m = jnp.maximum(tv, 1.0)
    i = lax.bitcast_convert_type(m, jnp.int32)
    i = jnp.int32(0x5F3759DF) - lax.shift_right_arithmetic(i, 1)
    y = lax.bitcast_convert_type(i, jnp.float32)
    h = 0.5 * m
    y = y * (1.5 - h * y * y)
    y = y * (1.5 - h * y * y)
    f = y * SCALE
    for k in range(4):
        out_v[r, pl.ds(16 * k, 16)] = vs[k] * f


@functools.partial(jax.jit, static_argnames=("b", "l"))
def _emb_lookup(tokens_flat, table2, *, b, l):
    n = b * l
    per_w = n // NW
    b_per_w = b // NW
    nchunk = b_per_w // BCHUNK
    chunk = BCHUNK * l

    mesh = plsc.VectorSubcoreMesh(core_axis_name="c", subcore_axis_name="s")

    @functools.partial(
        pl.kernel,
        mesh=mesh,
        out_type=jax.ShapeDtypeStruct((b, l, EMB), jnp.float32),
        scratch_types=[
            pltpu.VMEM((chunk + LANES,), jnp.int32),
            pltpu.VMEM((chunk + LANES,), jnp.int32),
            pltpu.VMEM((chunk,), jnp.int32),
            pltpu.VMEM((chunk, 2 * EMB), jnp.float32),
            pltpu.VMEM((chunk, 2 * EMB), jnp.float32),
            pltpu.VMEM((chunk, EMB), jnp.float32),
            pltpu.SemaphoreType.DMA,
        ],
    )
    def body(
        tok_hbm, table_hbm, out_hbm,
        tok_v0, tok_v1, idx_v, rows_v0, rows_v1, out_v, sem,
    ):
        wid = lax.axis_index("s") * NC + lax.axis_index("c")
        base = wid * per_w
        bbase = wid * b_per_w
        nidx = chunk // LANES + (1 if chunk % LANES else 0)
        toks = (tok_v0, tok_v1)
        rows = (rows_v0, rows_v1)

        def fetch(g, p):
            tok_v = toks[p]
            off = base + g * chunk
            pltpu.sync_copy(
                tok_hbm.at[pl.ds(off, chunk)], tok_v.at[pl.ds(0, chunk)]
            )

            @plsc.parallel_loop(0, nidx, unroll=4)
            def make_idx(i):
                s = jnp.minimum(i * LANES, chunk - LANES)
                t = tok_v[pl.ds(s, LANES)]
                idx_v[pl.ds(s, LANES)] = lax.shift_right_logical(t, 1)

            return pltpu.make_async_copy(
                table_hbm.at[idx_v], rows[p], sem
            )

        def consume(g, p):
            b0 = bbase + g * BCHUNK

            @plsc.parallel_loop(0, chunk, unroll=8)
            def do_row(r):
                _row_update(rows[p], out_v, toks[p], r)

            for bb in range(BCHUNK):
                pltpu.sync_copy(
                    out_v.at[pl.ds(bb * l, l)], out_hbm.at[b0 + bb]
                )

        fetch(0, 0).start()

        def do_pair(h, carry):
            for p in (0, 1):
                g = 2 * h + p
                pltpu.make_async_copy(table_hbm.at[idx_v], rows[p], sem).wait()
                gn = jnp.minimum(g + 1, nchunk - 1)
                fetch(gn, 1 - p).start()
                consume(g, p)
            return carry

        lax.fori_loop(0, nchunk // 2, do_pair, 0)
        pltpu.make_async_copy(table_hbm.at[idx_v], rows[0], sem).wait()

    return body(tokens_flat, table2)


def kernel(tokens, table):
    b, l = tokens.shape
    flat = tokens.reshape(-1).astype(jnp.int32)
    table2 = table.reshape(table.shape[0] // 2, 2 * EMB)
    return _emb_lookup(flat, table2, b=b, l=l)
